# baseline (speedup 1.0000x reference)
import functools

import jax
import jax.numpy as jnp
from jax import lax
from jax.experimental import pallas as pl
from jax.experimental.pallas import tpu as pltpu
from jax.experimental.pallas import tpu_sc as plsc

NC = 2
NS = 16
NW = NC * NS
EB = 128
NBUF = 1


@functools.lru_cache(maxsize=None)
def _make_sc_scatter(N_pad, D, NB):
    RPT = N_pad // NS
    HALF = NB // 2
    mesh = plsc.VectorSubcoreMesh(
        core_axis_name="c", subcore_axis_name="s", num_cores=NC, num_subcores=NS
    )

    @functools.partial(
        pl.kernel,
        mesh=mesh,
        out_type=jax.ShapeDtypeStruct((NC, N_pad, D), jnp.float32),
        scratch_types=[
            pltpu.VMEM((NB, EB), jnp.int32),
            pltpu.VMEM((NB, EB), jnp.int32),
            pltpu.VMEM((EB, D), jnp.float32),
            pltpu.VMEM_SHARED((N_pad, D), jnp.float32),
            pltpu.SemaphoreType.DMA,
        ],
    )
    def sc_kernel(x_hbm, src_hbm, dst_hbm, zeros_hbm, out_hbm,
                  src_v, dst_v, gbuf, agg_sh, sem):
        c = lax.axis_index("c")
        s = lax.axis_index("s")
        w = c * NS + s
        pltpu.sync_copy(zeros_hbm, agg_sh.at[pl.ds(s * RPT, RPT)])
        pltpu.sync_copy(src_hbm.at[w], src_v)
        pltpu.sync_copy(dst_hbm.at[w], dst_v)
        plsc.subcore_barrier()

        def body(j, carry):
            pltpu.async_copy(x_hbm.at[src_v.at[j]], gbuf, sem).wait()
            pltpu.sync_copy(gbuf, agg_sh.at[dst_v.at[j]], add=True)
            return carry

        lax.fori_loop(0, NB - 1, body, 0)
        body(NB - 1, 0)
        plsc.subcore_barrier()
        pltpu.sync_copy(agg_sh.at[pl.ds(s * RPT, RPT)],
                        out_hbm.at[c, pl.ds(s * RPT, RPT)])

    return sc_kernel


def _mlp_body(x_ref, agg_ref, eps_ref, w1_ref, b1_ref, w2_ref, b2_ref, out_ref):
    scale = 1.0 + eps_ref[0, 0]
    h = scale * x_ref[...] + agg_ref[0] + agg_ref[1]
    h = jnp.dot(h, w1_ref[...], preferred_element_type=jnp.float32) + b1_ref[...]
    h = jnp.maximum(h, 0.0)
    out_ref[...] = (
        jnp.dot(h, w2_ref[...], preferred_element_type=jnp.float32) + b2_ref[...]
    )


def kernel(x, edge_index, eps, W1, b1, W2, b2):
    N, D = x.shape
    E = edge_index.shape[1]

    ept = -(-E // NW)
    ept_pad = -(-ept // (EB * 16)) * (EB * 16)
    NB = ept_pad // EB
    E_pad = ept_pad * NW
    N_pad = -(-(N + 1) // (NS * 8)) * (NS * 8)

    src = edge_index[0]
    dst = edge_index[1]
    if E_pad != E:
        src = jnp.concatenate([src, jnp.zeros((E_pad - E,), jnp.int32)])
        dst = jnp.concatenate([dst, jnp.full((E_pad - E,), N, jnp.int32)])
    src_p = src.reshape(NW, NB, EB)
    dst_p = dst.reshape(NW, NB, EB)
    zeros = jnp.zeros((N_pad // NS, D), jnp.float32)

    agg2 = _make_sc_scatter(N_pad, D, NB)(x, src_p, dst_p, zeros)

    BR = next(b for b in (1000, 800, 500, 400, 250, 200, 125, 100, 50, 40,
                          25, 20, 10, 8, 5, 4, 2, 1) if N % b == 0)
    grid = (N // BR,)
    out = pl.pallas_call(
        _mlp_body,
        grid=grid,
        in_specs=[
            pl.BlockSpec((BR, D), lambda i: (i, 0)),
            pl.BlockSpec((NC, BR, D), lambda i: (0, i, 0)),
            pl.BlockSpec(memory_space=pltpu.SMEM),
            pl.BlockSpec((D, D), lambda i: (0, 0)),
            pl.BlockSpec((1, D), lambda i: (0, 0)),
            pl.BlockSpec((D, D), lambda i: (0, 0)),
            pl.BlockSpec((1, D), lambda i: (0, 0)),
        ],
        out_specs=pl.BlockSpec((BR, D), lambda i: (i, 0)),
        out_shape=jax.ShapeDtypeStruct((N, D), jnp.float32),
    )(x, agg2, eps.astype(jnp.float32).reshape(1, 1),
      W1, b1.reshape(1, D), W2, b2.reshape(1, D))
    return out


# speedup vs baseline: 1.0008x; 1.0008x over previous
import functools

import jax
import jax.numpy as jnp
from jax import lax
from jax.experimental import pallas as pl
from jax.experimental.pallas import tpu as pltpu
from jax.experimental.pallas import tpu_sc as plsc

NC = 2
NS = 16
NW = NC * NS
EB = 128
NBUF = 1


@functools.lru_cache(maxsize=None)
def _make_sc_scatter(N_pad, D, NB):
    RPT = N_pad // NS
    HALF = NB // 2
    mesh = plsc.VectorSubcoreMesh(
        core_axis_name="c", subcore_axis_name="s", num_cores=NC, num_subcores=NS
    )

    @functools.partial(
        pl.kernel,
        mesh=mesh,
        out_type=jax.ShapeDtypeStruct((NC, N_pad, D), jnp.float32),
        scratch_types=[
            pltpu.VMEM((NB, EB), jnp.int32),
            pltpu.VMEM((NB, EB), jnp.int32),
            pltpu.VMEM((EB, D), jnp.float32),
            pltpu.VMEM_SHARED((N_pad, D), jnp.float32),
            pltpu.SemaphoreType.DMA,
        ],
    )
    def sc_kernel(x_hbm, src_hbm, dst_hbm, zeros_hbm, out_hbm,
                  src_v, dst_v, gbuf, agg_sh, sem):
        c = lax.axis_index("c")
        s = lax.axis_index("s")
        w = c * NS + s
        pltpu.sync_copy(zeros_hbm, agg_sh.at[pl.ds(s * RPT, RPT)])
        pltpu.sync_copy(src_hbm.at[w], src_v)
        pltpu.sync_copy(dst_hbm.at[w], dst_v)
        plsc.subcore_barrier()

        def body(j, carry):
            pltpu.async_copy(x_hbm.at[src_v.at[j]], gbuf, sem).wait()
            pltpu.sync_copy(gbuf, agg_sh.at[dst_v.at[j]], add=True)
            return carry

        lax.fori_loop(0, NB - 1, body, 0)
        body(NB - 1, 0)
        plsc.subcore_barrier()
        pltpu.sync_copy(agg_sh.at[pl.ds(s * RPT, RPT)],
                        out_hbm.at[c, pl.ds(s * RPT, RPT)])

    return sc_kernel


def _mlp_body(x_ref, agg_ref, eps_ref, w1_ref, b1_ref, w2_ref, b2_ref, out_ref):
    scale = 1.0 + eps_ref[0, 0]
    h = scale * x_ref[...] + agg_ref[0] + agg_ref[1]
    h = jnp.dot(h, w1_ref[...], preferred_element_type=jnp.float32) + b1_ref[...]
    h = jnp.maximum(h, 0.0)
    out_ref[...] = (
        jnp.dot(h, w2_ref[...], preferred_element_type=jnp.float32) + b2_ref[...]
    )


def kernel(x, edge_index, eps, W1, b1, W2, b2):
    N, D = x.shape
    E = edge_index.shape[1]

    ept = -(-E // NW)
    ept_pad = -(-ept // (EB * 16)) * (EB * 16)
    NB = ept_pad // EB
    E_pad = ept_pad * NW
    N_pad = -(-(N + 1) // (NS * 8)) * (NS * 8)

    src = edge_index[0]
    dst = edge_index[1]
    if E_pad != E:
        pad_dst = N + jnp.arange(E_pad - E, dtype=jnp.int32) % (N_pad - N)
        src = jnp.concatenate([src, jnp.zeros((E_pad - E,), jnp.int32)])
        dst = jnp.concatenate([dst, pad_dst])
    src_p = src.reshape(NW, NB, EB)
    dst_p = dst.reshape(NW, NB, EB)
    zeros = jnp.zeros((N_pad // NS, D), jnp.float32)

    agg2 = _make_sc_scatter(N_pad, D, NB)(x, src_p, dst_p, zeros)

    BR = next(b for b in (1000, 800, 500, 400, 250, 200, 125, 100, 50, 40,
                          25, 20, 10, 8, 5, 4, 2, 1) if N % b == 0)
    grid = (N // BR,)
    out = pl.pallas_call(
        _mlp_body,
        grid=grid,
        in_specs=[
            pl.BlockSpec((BR, D), lambda i: (i, 0)),
            pl.BlockSpec((NC, BR, D), lambda i: (0, i, 0)),
            pl.BlockSpec(memory_space=pltpu.SMEM),
            pl.BlockSpec((D, D), lambda i: (0, 0)),
            pl.BlockSpec((1, D), lambda i: (0, 0)),
            pl.BlockSpec((D, D), lambda i: (0, 0)),
            pl.BlockSpec((1, D), lambda i: (0, 0)),
        ],
        out_specs=pl.BlockSpec((BR, D), lambda i: (i, 0)),
        out_shape=jax.ShapeDtypeStruct((N, D), jnp.float32),
    )(x, agg2, eps.astype(jnp.float32).reshape(1, 1),
      W1, b1.reshape(1, D), W2, b2.reshape(1, D))
    return out


# speedup vs baseline: 2.5246x; 2.5227x over previous
import functools

import jax
import jax.numpy as jnp
from jax import lax
from jax.experimental import pallas as pl
from jax.experimental.pallas import tpu as pltpu
from jax.experimental.pallas import tpu_sc as plsc

NC = 2
NS = 16
NW = NC * NS
EB = 128
NBUF = 1


@functools.lru_cache(maxsize=None)
def _make_sc_scatter(N_pad, D, NB):
    RPT = N_pad // NS
    HALF = NB // 2
    mesh = plsc.VectorSubcoreMesh(
        core_axis_name="c", subcore_axis_name="s", num_cores=NC, num_subcores=NS
    )

    @functools.partial(
        pl.kernel,
        mesh=mesh,
        out_type=jax.ShapeDtypeStruct((NC, N_pad, D), jnp.float32),
        scratch_types=[
            pltpu.VMEM((NB, EB), jnp.int32),
            pltpu.VMEM((NB, EB), jnp.int32),
            pltpu.VMEM((EB, D), jnp.float32),
            pltpu.VMEM_SHARED((N_pad, D), jnp.float32),
            pltpu.SemaphoreType.DMA,
        ],
    )
    def sc_kernel(x_hbm, src_hbm, dst_hbm, zeros_hbm, out_hbm,
                  src_v, dst_v, gbuf, agg_sh, sem):
        c = lax.axis_index("c")
        s = lax.axis_index("s")
        w = c * NS + s
        pltpu.sync_copy(zeros_hbm, agg_sh.at[pl.ds(s * RPT, RPT)])
        pltpu.sync_copy(src_hbm.at[w], src_v)
        pltpu.sync_copy(dst_hbm.at[w], dst_v)
        plsc.subcore_barrier()

        def body(j, carry):
            pltpu.async_copy(x_hbm.at[src_v.at[j]], gbuf, sem).wait()
            pltpu.sync_copy(gbuf, agg_sh.at[dst_v.at[j]], add=True)
            return carry

        lax.fori_loop(0, NB - 1, body, 0)
        body(NB - 1, 0)
        plsc.subcore_barrier()
        pltpu.sync_copy(agg_sh.at[pl.ds(s * RPT, RPT)],
                        out_hbm.at[c, pl.ds(s * RPT, RPT)])

    return sc_kernel


def _mlp_body(x_ref, agg_ref, eps_ref, w1_ref, b1_ref, w2_ref, b2_ref, out_ref):
    scale = 1.0 + eps_ref[0, 0]
    h = scale * x_ref[...] + agg_ref[0] + agg_ref[1]
    h = jnp.dot(h, w1_ref[...], preferred_element_type=jnp.float32) + b1_ref[...]
    h = jnp.maximum(h, 0.0)
    out_ref[...] = (
        jnp.dot(h, w2_ref[...], preferred_element_type=jnp.float32) + b2_ref[...]
    )


def kernel(x, edge_index, eps, W1, b1, W2, b2):
    N, D = x.shape
    E = edge_index.shape[1]

    ept = -(-E // NW)
    ept_pad = -(-ept // (EB * 16)) * (EB * 16)
    NB = ept_pad // EB
    E_pad = ept_pad * NW
    N_pad = -(-(N + 1) // (NS * 8)) * (NS * 8)

    src = edge_index[0]
    dst = edge_index[1]
    if E_pad != E:
        pad_ar = jnp.arange(E_pad - E, dtype=jnp.int32)
        pad_dst = N + pad_ar % (N_pad - N)
        pad_src = (pad_ar * 37) % N
        src = jnp.concatenate([src, pad_src])
        dst = jnp.concatenate([dst, pad_dst])
    src_p = src.reshape(NW, NB, EB)
    dst_p = dst.reshape(NW, NB, EB)
    zeros = jnp.zeros((N_pad // NS, D), jnp.float32)

    agg2 = _make_sc_scatter(N_pad, D, NB)(x, src_p, dst_p, zeros)

    BR = next(b for b in (1000, 800, 500, 400, 250, 200, 125, 100, 50, 40,
                          25, 20, 10, 8, 5, 4, 2, 1) if N % b == 0)
    grid = (N // BR,)
    out = pl.pallas_call(
        _mlp_body,
        grid=grid,
        in_specs=[
            pl.BlockSpec((BR, D), lambda i: (i, 0)),
            pl.BlockSpec((NC, BR, D), lambda i: (0, i, 0)),
            pl.BlockSpec(memory_space=pltpu.SMEM),
            pl.BlockSpec((D, D), lambda i: (0, 0)),
            pl.BlockSpec((1, D), lambda i: (0, 0)),
            pl.BlockSpec((D, D), lambda i: (0, 0)),
            pl.BlockSpec((1, D), lambda i: (0, 0)),
        ],
        out_specs=pl.BlockSpec((BR, D), lambda i: (i, 0)),
        out_shape=jax.ShapeDtypeStruct((N, D), jnp.float32),
    )(x, agg2, eps.astype(jnp.float32).reshape(1, 1),
      W1, b1.reshape(1, D), W2, b2.reshape(1, D))
    return out


# speedup vs baseline: 3.6139x; 1.4315x over previous
import functools

import jax
import jax.numpy as jnp
from jax import lax
from jax.experimental import pallas as pl
from jax.experimental.pallas import tpu as pltpu
from jax.experimental.pallas import tpu_sc as plsc

NC = 2
NS = 16
NW = NC * NS
EB = 128
NBUF = 2


@functools.lru_cache(maxsize=None)
def _make_sc_scatter(N_pad, D, NB):
    RPT = N_pad // NS
    HALF = NB // 2
    mesh = plsc.VectorSubcoreMesh(
        core_axis_name="c", subcore_axis_name="s", num_cores=NC, num_subcores=NS
    )

    @functools.partial(
        pl.kernel,
        mesh=mesh,
        out_type=jax.ShapeDtypeStruct((NC, N_pad, D), jnp.float32),
        scratch_types=[
            pltpu.VMEM((HALF, EB), jnp.int32),
            pltpu.VMEM((HALF, EB), jnp.int32),
            [pltpu.VMEM((EB, D), jnp.float32) for _ in range(NBUF)],
            pltpu.VMEM_SHARED((N_pad, D), jnp.float32),
            [pltpu.SemaphoreType.DMA for _ in range(NBUF)],
        ],
    )
    def sc_kernel(x_hbm, src_hbm, dst_hbm, zeros_hbm, out_hbm,
                  src_v, dst_v, gbufs, agg_sh, sems):
        c = lax.axis_index("c")
        s = lax.axis_index("s")
        w = c * NS + s

        def stage(h):
            pltpu.sync_copy(src_hbm.at[w, pl.ds(h * HALF, HALF)], src_v)
            pltpu.sync_copy(dst_hbm.at[w, pl.ds(h * HALF, HALF)], dst_v)

        def fire(j, b):
            pltpu.async_copy(x_hbm.at[src_v.at[j]], gbufs[b], sems[b])

        def drain_scatter(j, b):
            pltpu.make_async_copy(x_hbm.at[src_v.at[j]], gbufs[b], sems[b]).wait()
            pltpu.sync_copy(gbufs[b], agg_sh.at[dst_v.at[j]], add=True)

        def run_half():
            for b in range(NBUF):
                fire(b, b)

            def group(g, carry):
                for b in range(NBUF):
                    j = g * NBUF + b
                    drain_scatter(j, b)
                    fire(j + NBUF, b)
                return carry

            lax.fori_loop(0, HALF // NBUF - 1, group, 0)
            for b in range(NBUF):
                drain_scatter(HALF - NBUF + b, b)

        pltpu.sync_copy(zeros_hbm, agg_sh.at[pl.ds(s * RPT, RPT)])
        stage(0)
        plsc.subcore_barrier()
        run_half()
        stage(1)
        run_half()
        plsc.subcore_barrier()
        pltpu.sync_copy(agg_sh.at[pl.ds(s * RPT, RPT)],
                        out_hbm.at[c, pl.ds(s * RPT, RPT)])

    return sc_kernel


def _mlp_body(x_ref, agg_ref, eps_ref, w1_ref, b1_ref, w2_ref, b2_ref, out_ref):
    scale = 1.0 + eps_ref[0, 0]
    h = scale * x_ref[...] + agg_ref[0] + agg_ref[1]
    h = jnp.dot(h, w1_ref[...], preferred_element_type=jnp.float32) + b1_ref[...]
    h = jnp.maximum(h, 0.0)
    out_ref[...] = (
        jnp.dot(h, w2_ref[...], preferred_element_type=jnp.float32) + b2_ref[...]
    )


def kernel(x, edge_index, eps, W1, b1, W2, b2):
    N, D = x.shape
    E = edge_index.shape[1]

    ept = -(-E // NW)
    ept_pad = -(-ept // (EB * 16)) * (EB * 16)
    NB = ept_pad // EB
    E_pad = ept_pad * NW
    N_pad = -(-(N + 1) // (NS * 8)) * (NS * 8)

    src = edge_index[0]
    dst = edge_index[1]
    if E_pad != E:
        pad_ar = jnp.arange(E_pad - E, dtype=jnp.int32)
        pad_dst = N + pad_ar % (N_pad - N)
        pad_src = (pad_ar * 37) % N
        src = jnp.concatenate([src, pad_src])
        dst = jnp.concatenate([dst, pad_dst])
    src_p = src.reshape(NW, NB, EB)
    dst_p = dst.reshape(NW, NB, EB)
    zeros = jnp.zeros((N_pad // NS, D), jnp.float32)

    agg2 = _make_sc_scatter(N_pad, D, NB)(x, src_p, dst_p, zeros)

    BR = next(b for b in (1000, 800, 500, 400, 250, 200, 125, 100, 50, 40,
                          25, 20, 10, 8, 5, 4, 2, 1) if N % b == 0)
    grid = (N // BR,)
    out = pl.pallas_call(
        _mlp_body,
        grid=grid,
        in_specs=[
            pl.BlockSpec((BR, D), lambda i: (i, 0)),
            pl.BlockSpec((NC, BR, D), lambda i: (0, i, 0)),
            pl.BlockSpec(memory_space=pltpu.SMEM),
            pl.BlockSpec((D, D), lambda i: (0, 0)),
            pl.BlockSpec((1, D), lambda i: (0, 0)),
            pl.BlockSpec((D, D), lambda i: (0, 0)),
            pl.BlockSpec((1, D), lambda i: (0, 0)),
        ],
        out_specs=pl.BlockSpec((BR, D), lambda i: (i, 0)),
        out_shape=jax.ShapeDtypeStruct((N, D), jnp.float32),
    )(x, agg2, eps.astype(jnp.float32).reshape(1, 1),
      W1, b1.reshape(1, D), W2, b2.reshape(1, D))
    return out
